# baseline (speedup 1.0000x reference)
import functools

import jax
import jax.numpy as jnp
from jax import lax
from jax.experimental import pallas as pl
from jax.experimental.pallas import tpu as pltpu
from jax.experimental.pallas import tpu_sc as plsc

_CHUNK = 64
_NSUB = 16
_LANES = 16
_NBUF = 4


@functools.lru_cache(maxsize=None)
def _make_sc_spmm(n_nodes, d_feat, n_chunks_per_tile):
  assert n_chunks_per_tile % _NBUF == 0
  mesh = plsc.VectorSubcoreMesh(core_axis_name="c", subcore_axis_name="s")
  rows_per_tile = n_nodes // _NSUB
  last = n_chunks_per_tile - 1

  @functools.partial(
      pl.kernel,
      mesh=mesh,
      out_type=jax.ShapeDtypeStruct((2, n_nodes, d_feat), jnp.float32),
      compiler_params=pltpu.CompilerParams(
          use_tc_tiling_on_sc=False, needs_layout_passes=False),
      scratch_types=[
          pltpu.VMEM_SHARED((n_nodes, d_feat), jnp.float32),
          [pltpu.VMEM((3, _CHUNK), jnp.int32) for _ in range(_NBUF)],
          [pltpu.VMEM((_CHUNK, d_feat), jnp.float32) for _ in range(_NBUF)],
          [pltpu.SemaphoreType.DMA for _ in range(_NBUF)],
          [pltpu.SemaphoreType.DMA for _ in range(_NBUF)],
          [pltpu.SemaphoreType.DMA for _ in range(_NBUF)],
      ],
  )
  def spmm(x_hbm, pk_hbm, zero_hbm, out_hbm,
           acc, idx, msg, isem, gsem, ssem):
    c = lax.axis_index("c")
    s = lax.axis_index("s")

    r0 = s * rows_per_tile
    pltpu.sync_copy(zero_hbm.at[pl.ds(r0, rows_per_tile)],
                    acc.at[pl.ds(r0, rows_per_tile)])
    plsc.subcore_barrier()

    def load_idx(g, b):
      pltpu.async_copy(pk_hbm.at[c, s, g], idx[b], isem[b])

    def start_gather(b):
      pltpu.async_copy(x_hbm.at[idx[b].at[1]], msg[b], gsem[b])

    def scale(b):
      mref = msg[b]
      iref = idx[b]
      dnums = lax.GatherDimensionNumbers(
          offset_dims=(), collapsed_slice_dims=(0,), start_index_map=(0,))

      def ubody(u, _):
        a = plsc.bitcast(iref[2, pl.ds(u * _LANES, _LANES)], jnp.float32)
        for kk in range(_LANES):
          av = lax.gather(a, jnp.full((_LANES, 1), kk, jnp.int32), dnums,
                          slice_sizes=(1,),
                          mode=lax.GatherScatterMode.PROMISE_IN_BOUNDS)
          k = u * _LANES + kk
          for j in range(d_feat // _LANES):
            sl = pl.ds(j * _LANES, _LANES)
            mref[k, sl] = mref[k, sl] * av
        return 0

      lax.fori_loop(0, _CHUNK // _LANES, ubody, 0)

    load_idx(0, 0)
    load_idx(1, 1)
    pltpu.make_async_copy(pk_hbm.at[c, s, 0], idx[0], isem[0]).wait()
    start_gather(0)

    def ring_body(i, _):
      for bb in range(_NBUF):
        g = _NBUF * i + bb
        b = bb
        b1 = (bb + 1) % _NBUF
        b2 = (bb + 2) % _NBUF

        @pl.when(g >= 2)
        def _():
          pltpu.make_async_copy(msg[b2], acc.at[idx[b2].at[0]],
                                ssem[b2]).wait()

        @pl.when(g + 2 <= last)
        def _():
          load_idx(g + 2, b2)

        @pl.when(g + 1 <= last)
        def _():
          pltpu.make_async_copy(pk_hbm.at[c, s, g + 1], idx[b1],
                                isem[b1]).wait()
          start_gather(b1)

        pltpu.make_async_copy(x_hbm.at[idx[b].at[1]], msg[b], gsem[b]).wait()
        scale(b)
        pltpu.async_copy(msg[b], acc.at[idx[b].at[0]], ssem[b], add=True)
      return 0

    lax.fori_loop(0, n_chunks_per_tile // _NBUF, ring_body, 0)
    for g in (last - 1, last):
      b = g % _NBUF
      pltpu.make_async_copy(msg[b], acc.at[idx[b].at[0]], ssem[b]).wait()

    plsc.subcore_barrier()
    pltpu.sync_copy(acc.at[pl.ds(r0, rows_per_tile)],
                    out_hbm.at[c, pl.ds(r0, rows_per_tile)])

  return spmm


@functools.lru_cache(maxsize=None)
def _make_combine(n_nodes, d_feat, blk):
  def body(p_ref, o_ref):
    o_ref[...] = p_ref[0] + p_ref[1]

  return pl.pallas_call(
      body,
      grid=(n_nodes // blk,),
      in_specs=[pl.BlockSpec((2, blk, d_feat), lambda i: (0, i, 0))],
      out_specs=pl.BlockSpec((blk, d_feat), lambda i: (i, 0)),
      out_shape=jax.ShapeDtypeStruct((n_nodes, d_feat), jnp.float32),
  )


def kernel(x, edge_index, edge_attr):
  n, d = x.shape
  e = edge_attr.shape[0]
  row = edge_index[0].astype(jnp.int32)
  col = edge_index[1].astype(jnp.int32)
  attr_bits = jax.lax.bitcast_convert_type(
      edge_attr.astype(jnp.float32), jnp.int32)

  n_chunks_per_tile = -(-e // (2 * _NSUB * _CHUNK))
  n_chunks_per_tile += (-n_chunks_per_tile) % _NBUF
  e_pad = n_chunks_per_tile * 2 * _NSUB * _CHUNK
  pad = e_pad - e
  if pad:
    row = jnp.concatenate([row, jnp.zeros((pad,), jnp.int32)])
    col = jnp.concatenate([col, jnp.zeros((pad,), jnp.int32)])
    attr_bits = jnp.concatenate([attr_bits, jnp.zeros((pad,), jnp.int32)])

  shp = (2, _NSUB, n_chunks_per_tile, _CHUNK)
  pk = jnp.stack([row.reshape(shp), col.reshape(shp), attr_bits.reshape(shp)],
                 axis=3)

  zero = jnp.zeros((n, d), jnp.float32)
  partial = _make_sc_spmm(n, d, n_chunks_per_tile)(x, pk, zero)
  return _make_combine(n, d, 1000)(partial)


# speedup vs baseline: 2.7843x; 2.7843x over previous
import functools

import jax
import jax.numpy as jnp
from jax import lax
from jax.experimental import pallas as pl
from jax.experimental.pallas import tpu as pltpu
from jax.experimental.pallas import tpu_sc as plsc

_CHUNK = 64
_NSUB = 16
_LANES = 16
_NBUF = 4


@functools.lru_cache(maxsize=None)
def _make_sc_spmm(n_nodes, d_feat, n_chunks_per_tile):
  assert n_chunks_per_tile % _NBUF == 0
  mesh = plsc.VectorSubcoreMesh(core_axis_name="c", subcore_axis_name="s")
  rows_per_tile = n_nodes // _NSUB
  last = n_chunks_per_tile - 1

  @functools.partial(
      pl.kernel,
      mesh=mesh,
      out_type=jax.ShapeDtypeStruct((2, n_nodes, d_feat), jnp.float32),
      compiler_params=pltpu.CompilerParams(
          use_tc_tiling_on_sc=False, needs_layout_passes=False),
      scratch_types=[
          pltpu.VMEM_SHARED((n_nodes, d_feat), jnp.float32),
          [pltpu.VMEM((3, _CHUNK), jnp.int32) for _ in range(_NBUF)],
          [pltpu.VMEM((_CHUNK, d_feat), jnp.float32) for _ in range(_NBUF)],
          [pltpu.SemaphoreType.DMA for _ in range(_NBUF)],
          [pltpu.SemaphoreType.DMA for _ in range(_NBUF)],
          [pltpu.SemaphoreType.DMA for _ in range(_NBUF)],
      ],
  )
  def spmm(x_hbm, pk_hbm, zero_hbm, out_hbm,
           acc, idx, msg, isem, gsem, ssem):
    c = lax.axis_index("c")
    s = lax.axis_index("s")

    r0 = s * rows_per_tile
    pltpu.sync_copy(zero_hbm.at[pl.ds(r0, rows_per_tile)],
                    acc.at[pl.ds(r0, rows_per_tile)])
    plsc.subcore_barrier()

    def load_idx(g, b):
      pltpu.async_copy(pk_hbm.at[c, s, g], idx[b], isem[b])

    def start_gather(b):
      pltpu.async_copy(x_hbm.at[idx[b].at[1]], msg[b], gsem[b])

    def scale(b):
      mref = msg[b]
      iref = idx[b]
      dnums = lax.GatherDimensionNumbers(
          offset_dims=(), collapsed_slice_dims=(0,), start_index_map=(0,))

      def ubody(u, _):
        a = plsc.bitcast(iref[2, pl.ds(u * _LANES, _LANES)], jnp.float32)
        for kk in range(_LANES):
          av = lax.gather(a, jnp.full((_LANES, 1), kk, jnp.int32), dnums,
                          slice_sizes=(1,),
                          mode=lax.GatherScatterMode.PROMISE_IN_BOUNDS)
          k = u * _LANES + kk
          for j in range(d_feat // _LANES):
            sl = pl.ds(j * _LANES, _LANES)
            mref[k, sl] = mref[k, sl] * av
        return 0

      lax.fori_loop(0, _CHUNK // _LANES, ubody, 0)

    load_idx(0, 0)
    load_idx(1, 1)
    pltpu.make_async_copy(pk_hbm.at[c, s, 0], idx[0], isem[0]).wait()
    start_gather(0)

    def ring_body(i, _):
      for bb in range(_NBUF):
        g = _NBUF * i + bb
        b = bb
        b1 = (bb + 1) % _NBUF
        b2 = (bb + 2) % _NBUF

        @pl.when(g >= 2)
        def _():
          pltpu.make_async_copy(msg[b2], acc.at[idx[b2].at[0]],
                                ssem[b2]).wait()

        @pl.when(g + 2 <= last)
        def _():
          load_idx(g + 2, b2)

        @pl.when(g + 1 <= last)
        def _():
          pltpu.make_async_copy(pk_hbm.at[c, s, g + 1], idx[b1],
                                isem[b1]).wait()
          start_gather(b1)

        pltpu.make_async_copy(x_hbm.at[idx[b].at[1]], msg[b], gsem[b]).wait()
        scale(b)
        pltpu.async_copy(msg[b], acc.at[idx[b].at[0]], ssem[b], add=True)
      return 0

    lax.fori_loop(0, n_chunks_per_tile // _NBUF, ring_body, 0)
    for g in (last - 1, last):
      b = g % _NBUF
      pltpu.make_async_copy(msg[b], acc.at[idx[b].at[0]], ssem[b]).wait()

    plsc.subcore_barrier()
    pltpu.sync_copy(acc.at[pl.ds(r0, rows_per_tile)],
                    out_hbm.at[c, pl.ds(r0, rows_per_tile)])

  return spmm


@functools.lru_cache(maxsize=None)
def _make_combine(n_nodes, d_feat, blk):
  def body(p_ref, o_ref):
    o_ref[...] = p_ref[0] + p_ref[1]

  return pl.pallas_call(
      body,
      grid=(n_nodes // blk,),
      in_specs=[pl.BlockSpec((2, blk, d_feat), lambda i: (0, i, 0))],
      out_specs=pl.BlockSpec((blk, d_feat), lambda i: (i, 0)),
      out_shape=jax.ShapeDtypeStruct((n_nodes, d_feat), jnp.float32),
  )


def kernel(x, edge_index, edge_attr):
  n, d = x.shape
  e = edge_attr.shape[0]
  row = edge_index[0].astype(jnp.int32)
  col = edge_index[1].astype(jnp.int32)
  attr_bits = jax.lax.bitcast_convert_type(
      edge_attr.astype(jnp.float32), jnp.int32)

  n_chunks_per_tile = -(-e // (2 * _NSUB * _CHUNK))
  n_chunks_per_tile += (-n_chunks_per_tile) % _NBUF
  e_pad = n_chunks_per_tile * 2 * _NSUB * _CHUNK
  pad = e_pad - e
  if pad:
    spread = jnp.arange(pad, dtype=jnp.int32) % n
    row = jnp.concatenate([row, spread])
    col = jnp.concatenate([col, spread])
    attr_bits = jnp.concatenate([attr_bits, jnp.zeros((pad,), jnp.int32)])

  shp = (2, _NSUB, n_chunks_per_tile, _CHUNK)
  pk = jnp.stack([row.reshape(shp), col.reshape(shp), attr_bits.reshape(shp)],
                 axis=3)

  zero = jnp.zeros((n, d), jnp.float32)
  partial = _make_sc_spmm(n, d, n_chunks_per_tile)(x, pk, zero)
  return _make_combine(n, d, 1000)(partial)


# speedup vs baseline: 2.9876x; 1.0730x over previous
import functools

import jax
import jax.numpy as jnp
from jax import lax
from jax.experimental import pallas as pl
from jax.experimental.pallas import tpu as pltpu
from jax.experimental.pallas import tpu_sc as plsc

_CHUNK = 64
_NSUB = 16
_LANES = 16
_NBUF = 4
_NIDX = 8


@functools.lru_cache(maxsize=None)
def _make_sc_spmm(n_nodes, d_feat, n_chunks_per_tile):
  assert n_chunks_per_tile % _NIDX == 0
  mesh = plsc.VectorSubcoreMesh(core_axis_name="c", subcore_axis_name="s")
  rows_per_tile = n_nodes // _NSUB
  last = n_chunks_per_tile - 1

  @functools.partial(
      pl.kernel,
      mesh=mesh,
      out_type=jax.ShapeDtypeStruct((2, n_nodes, d_feat), jnp.float32),
      compiler_params=pltpu.CompilerParams(
          use_tc_tiling_on_sc=False, needs_layout_passes=False),
      scratch_types=[
          pltpu.VMEM_SHARED((n_nodes, d_feat), jnp.float32),
          [pltpu.VMEM((3, _CHUNK), jnp.int32) for _ in range(_NIDX)],
          [pltpu.VMEM((_CHUNK, d_feat), jnp.float32) for _ in range(_NBUF)],
          [pltpu.SemaphoreType.DMA for _ in range(_NIDX)],
          [pltpu.SemaphoreType.DMA for _ in range(_NBUF)],
          [pltpu.SemaphoreType.DMA for _ in range(_NBUF)],
      ],
  )
  def spmm(x_hbm, pk_hbm, zero_hbm, out_hbm,
           acc, idx, msg, isem, gsem, ssem):
    c = lax.axis_index("c")
    s = lax.axis_index("s")

    r0 = s * rows_per_tile
    pltpu.sync_copy(zero_hbm.at[pl.ds(r0, rows_per_tile)],
                    acc.at[pl.ds(r0, rows_per_tile)])
    plsc.subcore_barrier()

    def load_idx(g, ib):
      pltpu.async_copy(pk_hbm.at[c, s, g], idx[ib], isem[ib])

    def start_gather(ib, mb):
      pltpu.async_copy(x_hbm.at[idx[ib].at[1]], msg[mb], gsem[mb])

    def scale(ib, mb):
      mref = msg[mb]
      iref = idx[ib]
      dnums = lax.GatherDimensionNumbers(
          offset_dims=(), collapsed_slice_dims=(0,), start_index_map=(0,))

      def ubody(u, _):
        a = plsc.bitcast(iref[2, pl.ds(u * _LANES, _LANES)], jnp.float32)
        for kk in range(_LANES):
          av = lax.gather(a, jnp.full((_LANES, 1), kk, jnp.int32), dnums,
                          slice_sizes=(1,),
                          mode=lax.GatherScatterMode.PROMISE_IN_BOUNDS)
          k = u * _LANES + kk
          for j in range(d_feat // _LANES):
            sl = pl.ds(j * _LANES, _LANES)
            mref[k, sl] = mref[k, sl] * av
        return 0

      lax.fori_loop(0, _CHUNK // _LANES, ubody, 0)

    for g0 in range(4):
      load_idx(g0, g0)
    for g0 in range(2):
      pltpu.make_async_copy(pk_hbm.at[c, s, g0], idx[g0], isem[g0]).wait()
      start_gather(g0, g0)

    def ring_body(i, _):
      for bb in range(_NIDX):
        g = _NIDX * i + bb
        mb = bb % _NBUF
        mb2 = (bb + 2) % _NBUF
        ib2 = (bb + 2) % _NIDX
        ib4 = (bb + 4) % _NIDX

        @pl.when(g >= 2)
        def _():
          pltpu.make_async_copy(msg[mb2], acc.at[idx[ib2].at[0]],
                                ssem[mb2]).wait()

        @pl.when(g + 4 <= last)
        def _():
          load_idx(g + 4, ib4)

        @pl.when(g + 2 <= last)
        def _():
          pltpu.make_async_copy(pk_hbm.at[c, s, g + 2], idx[ib2],
                                isem[ib2]).wait()
          start_gather(ib2, mb2)

        pltpu.make_async_copy(x_hbm.at[idx[bb].at[1]], msg[mb],
                              gsem[mb]).wait()
        scale(bb, mb)
        pltpu.async_copy(msg[mb], acc.at[idx[bb].at[0]], ssem[mb], add=True)
      return 0

    lax.fori_loop(0, n_chunks_per_tile // _NIDX, ring_body, 0)
    for g in (last - 1, last):
      pltpu.make_async_copy(msg[g % _NBUF], acc.at[idx[g % _NIDX].at[0]],
                            ssem[g % _NBUF]).wait()

    plsc.subcore_barrier()
    pltpu.sync_copy(acc.at[pl.ds(r0, rows_per_tile)],
                    out_hbm.at[c, pl.ds(r0, rows_per_tile)])

  return spmm


@functools.lru_cache(maxsize=None)
def _make_combine(n_nodes, d_feat, blk):
  def body(p_ref, o_ref):
    o_ref[...] = p_ref[0] + p_ref[1]

  return pl.pallas_call(
      body,
      grid=(n_nodes // blk,),
      in_specs=[pl.BlockSpec((2, blk, d_feat), lambda i: (0, i, 0))],
      out_specs=pl.BlockSpec((blk, d_feat), lambda i: (i, 0)),
      out_shape=jax.ShapeDtypeStruct((n_nodes, d_feat), jnp.float32),
  )


def kernel(x, edge_index, edge_attr):
  n, d = x.shape
  e = edge_attr.shape[0]
  row = edge_index[0].astype(jnp.int32)
  col = edge_index[1].astype(jnp.int32)
  attr_bits = jax.lax.bitcast_convert_type(
      edge_attr.astype(jnp.float32), jnp.int32)

  n_chunks_per_tile = -(-e // (2 * _NSUB * _CHUNK))
  n_chunks_per_tile += (-n_chunks_per_tile) % _NIDX
  e_pad = n_chunks_per_tile * 2 * _NSUB * _CHUNK
  pad = e_pad - e
  if pad:
    spread = jnp.arange(pad, dtype=jnp.int32) % n
    row = jnp.concatenate([row, spread])
    col = jnp.concatenate([col, spread])
    attr_bits = jnp.concatenate([attr_bits, jnp.zeros((pad,), jnp.int32)])

  shp = (2, _NSUB, n_chunks_per_tile, _CHUNK)
  pk = jnp.stack([row.reshape(shp), col.reshape(shp), attr_bits.reshape(shp)],
                 axis=3)

  zero = jnp.zeros((n, d), jnp.float32)
  partial = _make_sc_spmm(n, d, n_chunks_per_tile)(x, pk, zero)
  return _make_combine(n, d, 1000)(partial)


# speedup vs baseline: 2.9930x; 1.0018x over previous
import functools

import jax
import jax.numpy as jnp
from jax import lax
from jax.experimental import pallas as pl
from jax.experimental.pallas import tpu as pltpu
from jax.experimental.pallas import tpu_sc as plsc

_CHUNK = 64
_NSUB = 16
_LANES = 16
_NBUF = 4
_NIDX = 8
_LOOK = _NBUF // 2


@functools.lru_cache(maxsize=None)
def _make_sc_spmm(n_nodes, d_feat, n_chunks_per_tile):
  assert n_chunks_per_tile % _NIDX == 0
  mesh = plsc.VectorSubcoreMesh(core_axis_name="c", subcore_axis_name="s")
  rows_per_tile = n_nodes // _NSUB
  last = n_chunks_per_tile - 1

  @functools.partial(
      pl.kernel,
      mesh=mesh,
      out_type=jax.ShapeDtypeStruct((2, n_nodes, d_feat), jnp.float32),
      compiler_params=pltpu.CompilerParams(
          use_tc_tiling_on_sc=False, needs_layout_passes=False),
      scratch_types=[
          pltpu.VMEM_SHARED((n_nodes, d_feat), jnp.float32),
          [pltpu.VMEM((3, _CHUNK), jnp.int32) for _ in range(_NIDX)],
          [pltpu.VMEM((_CHUNK, d_feat), jnp.float32) for _ in range(_NBUF)],
          [pltpu.SemaphoreType.DMA for _ in range(_NIDX)],
          [pltpu.SemaphoreType.DMA for _ in range(_NBUF)],
          [pltpu.SemaphoreType.DMA for _ in range(_NBUF)],
      ],
  )
  def spmm(x_hbm, pk_hbm, zero_hbm, out_hbm,
           acc, idx, msg, isem, gsem, ssem):
    c = lax.axis_index("c")
    s = lax.axis_index("s")

    r0 = s * rows_per_tile
    pltpu.sync_copy(zero_hbm.at[pl.ds(r0, rows_per_tile)],
                    acc.at[pl.ds(r0, rows_per_tile)])
    plsc.subcore_barrier()

    def load_idx(g, ib):
      pltpu.async_copy(pk_hbm.at[c, s, g], idx[ib], isem[ib])

    def start_gather(ib, mb):
      pltpu.async_copy(x_hbm.at[idx[ib].at[1]], msg[mb], gsem[mb])

    def scale(ib, mb):
      mref = msg[mb]
      iref = idx[ib]
      dnums = lax.GatherDimensionNumbers(
          offset_dims=(), collapsed_slice_dims=(0,), start_index_map=(0,))

      def ubody(u, _):
        a = plsc.bitcast(iref[2, pl.ds(u * _LANES, _LANES)], jnp.float32)
        for kk in range(_LANES):
          av = lax.gather(a, jnp.full((_LANES, 1), kk, jnp.int32), dnums,
                          slice_sizes=(1,),
                          mode=lax.GatherScatterMode.PROMISE_IN_BOUNDS)
          k = u * _LANES + kk
          for j in range(d_feat // _LANES):
            sl = pl.ds(j * _LANES, _LANES)
            mref[k, sl] = mref[k, sl] * av
        return 0

      lax.fori_loop(0, _CHUNK // _LANES, ubody, 0)

    for g0 in range(2 * _LOOK):
      load_idx(g0, g0)
    for g0 in range(_LOOK):
      pltpu.make_async_copy(pk_hbm.at[c, s, g0], idx[g0], isem[g0]).wait()
      start_gather(g0, g0)

    def ring_body(i, _):
      for bb in range(_NIDX):
        g = _NIDX * i + bb
        mb = bb % _NBUF
        mbL = (bb + _LOOK) % _NBUF
        ibL = (bb + _LOOK) % _NIDX
        ib2L = (bb + 2 * _LOOK) % _NIDX

        @pl.when(g >= _LOOK)
        def _():
          pltpu.make_async_copy(msg[mbL], acc.at[idx[ibL].at[0]],
                                ssem[mbL]).wait()

        @pl.when(g + 2 * _LOOK <= last)
        def _():
          load_idx(g + 2 * _LOOK, ib2L)

        @pl.when(g + _LOOK <= last)
        def _():
          pltpu.make_async_copy(pk_hbm.at[c, s, g + _LOOK], idx[ibL],
                                isem[ibL]).wait()
          start_gather(ibL, mbL)

        pltpu.make_async_copy(x_hbm.at[idx[bb].at[1]], msg[mb],
                              gsem[mb]).wait()
        scale(bb, mb)
        pltpu.async_copy(msg[mb], acc.at[idx[bb].at[0]], ssem[mb], add=True)
      return 0

    lax.fori_loop(0, n_chunks_per_tile // _NIDX, ring_body, 0)
    for g in range(last - _LOOK + 1, last + 1):
      pltpu.make_async_copy(msg[g % _NBUF], acc.at[idx[g % _NIDX].at[0]],
                            ssem[g % _NBUF]).wait()

    plsc.subcore_barrier()
    pltpu.sync_copy(acc.at[pl.ds(r0, rows_per_tile)],
                    out_hbm.at[c, pl.ds(r0, rows_per_tile)])

  return spmm


@functools.lru_cache(maxsize=None)
def _make_combine(n_nodes, d_feat, blk):
  def body(p_ref, o_ref):
    o_ref[...] = p_ref[0] + p_ref[1]

  return pl.pallas_call(
      body,
      grid=(n_nodes // blk,),
      in_specs=[pl.BlockSpec((2, blk, d_feat), lambda i: (0, i, 0))],
      out_specs=pl.BlockSpec((blk, d_feat), lambda i: (i, 0)),
      out_shape=jax.ShapeDtypeStruct((n_nodes, d_feat), jnp.float32),
  )


def kernel(x, edge_index, edge_attr):
  n, d = x.shape
  e = edge_attr.shape[0]
  row = edge_index[0].astype(jnp.int32)
  col = edge_index[1].astype(jnp.int32)
  attr_bits = jax.lax.bitcast_convert_type(
      edge_attr.astype(jnp.float32), jnp.int32)

  n_chunks_per_tile = -(-e // (2 * _NSUB * _CHUNK))
  n_chunks_per_tile += (-n_chunks_per_tile) % _NIDX
  e_pad = n_chunks_per_tile * 2 * _NSUB * _CHUNK
  pad = e_pad - e
  if pad:
    spread = jnp.arange(pad, dtype=jnp.int32) % n
    row = jnp.concatenate([row, spread])
    col = jnp.concatenate([col, spread])
    attr_bits = jnp.concatenate([attr_bits, jnp.zeros((pad,), jnp.int32)])

  shp = (2, _NSUB, n_chunks_per_tile, _CHUNK)
  pk = jnp.stack([row.reshape(shp), col.reshape(shp), attr_bits.reshape(shp)],
                 axis=3)

  zero = jnp.zeros((n, d), jnp.float32)
  partial = _make_sc_spmm(n, d, n_chunks_per_tile)(x, pk, zero)
  return _make_combine(n, d, 1000)(partial)
